# Initial kernel scaffold; baseline (speedup 1.0000x reference)
import jax
import jax.numpy as jnp
from jax.experimental import pallas as pl


def kernel(time, abundances, temperature, cr_rate, fuv_rate, alpha, beta, gamma, zeta, xi, inc_vals, pair_reac, pair_species, inc_rows, inc_cols):
    raise NotImplementedError("write your pallas kernel here")


# speedup vs baseline: 200.3260x; 200.3260x over previous
import jax
import jax.numpy as jnp
from jax import lax
from jax.experimental import pallas as pl
from jax.experimental.pallas import tpu as pltpu
from jax.experimental.pallas import tpu_sc as plsc

NC = 2
NS = 16
NW = NC * NS
L = 16

NSPEC = 20000
S_PAD = 20480
SROW = S_PAD // L
RCH = SROW // 128
ORB = SROW // NS
NREAC = 200000
CH = 6272
R_PAD = NW * CH


def _sc_body(alpha_h, beta_h, gamma_h, zeta_h, xi_h,
             r1_h, r2_h, p1_h, p2_h, ab_h, zeros_h, iota_h, consts_h,
             out_h,
             ab_v, acc_v, al_v, be_v, ga_v, ze_v, xj_v,
             i1_v, i2_v, q1_v, q2_v, iot_v, cv, shared):
    c = lax.axis_index("c")
    s = lax.axis_index("s")
    wid = s * NC + c
    base = wid * CH

    @pl.when(s == 0)
    def _zero_shared():
        pltpu.sync_copy(zeros_h, shared)

    pltpu.sync_copy(ab_h, ab_v)
    pltpu.sync_copy(zeros_h, acc_v)
    pltpu.sync_copy(iota_h, iot_v)
    pltpu.sync_copy(consts_h, cv)
    pltpu.sync_copy(alpha_h.at[pl.ds(base, CH)], al_v)
    pltpu.sync_copy(beta_h.at[pl.ds(base, CH)], be_v)
    pltpu.sync_copy(gamma_h.at[pl.ds(base, CH)], ga_v)
    pltpu.sync_copy(zeta_h.at[pl.ds(base, CH)], ze_v)
    pltpu.sync_copy(xi_h.at[pl.ds(base, CH)], xj_v)
    pltpu.sync_copy(r1_h.at[pl.ds(base, CH)], i1_v)
    pltpu.sync_copy(r2_h.at[pl.ds(base, CH)], i2_v)
    pltpu.sync_copy(p1_h.at[pl.ds(base, CH)], q1_v)
    pltpu.sync_copy(p2_h.at[pl.ds(base, CH)], q2_v)

    c1 = cv[pl.ds(0, L)]
    c2 = cv[pl.ds(L, L)]
    crv = cv[pl.ds(2 * L, L)]
    fuv = cv[pl.ds(3 * L, L)]

    def body(j, carry):
        o = j * L
        a = al_v[pl.ds(o, L)]
        b = be_v[pl.ds(o, L)]
        g = ga_v[pl.ds(o, L)]
        z = ze_v[pl.ds(o, L)]
        x = xj_v[pl.ds(o, L)]
        i1 = i1_v[pl.ds(o, L)]
        i2 = i2_v[pl.ds(o, L)]
        q1 = q1_v[pl.ds(o, L)]
        q2 = q2_v[pl.ds(o, L)]
        rate = a * jnp.exp(b * c1 - g * c2) + z * crv + x * fuv
        ab1 = plsc.load_gather(ab_v, [i1])
        ab2 = plsc.load_gather(ab_v, [i2])
        rate = rate * ab1 * ab2
        neg = -rate
        plsc.addupdate_scatter(acc_v, [q1 >> 4, q1 & 15], rate)
        plsc.addupdate_scatter(acc_v, [q2 >> 4, q2 & 15], rate)
        plsc.addupdate_scatter(acc_v, [i1 >> 4, i1 & 15], neg)
        plsc.addupdate_scatter(acc_v, [i2 >> 4, i2 & 15], neg)
        return carry

    lax.fori_loop(0, CH // L, body, 0)

    plsc.subcore_barrier()
    for j in range(RCH):
        pltpu.sync_copy(acc_v.at[pl.ds(j * 128, 128)],
                        shared.at[iot_v.at[j]], add=True)
    plsc.subcore_barrier()
    pltpu.sync_copy(shared.at[pl.ds(s * ORB, ORB)],
                    out_h.at[c].at[pl.ds(s * ORB, ORB)])


def _combine_body(x_ref, o_ref):
    o_ref[...] = x_ref[0] + x_ref[1]


def kernel(time, abundances, temperature, cr_rate, fuv_rate, alpha, beta,
           gamma, zeta, xi, inc_vals, pair_reac, pair_species, inc_rows,
           inc_cols):
    f32 = jnp.float32
    r = alpha.shape[0]
    pad = R_PAD - r
    zf = jnp.zeros((pad,), f32)
    zi = jnp.zeros((pad,), jnp.int32)
    alpha_p = jnp.concatenate([alpha, zf])
    beta_p = jnp.concatenate([beta, zf])
    gamma_p = jnp.concatenate([gamma, zf])
    zeta_p = jnp.concatenate([zeta, zf])
    xi_p = jnp.concatenate([xi, zf])
    r1 = jnp.concatenate([pair_species[:r], zi])
    r2 = jnp.concatenate([pair_species[r:], zi])
    p1 = jnp.concatenate([inc_rows[2 * r:3 * r], zi])
    p2 = jnp.concatenate([inc_rows[3 * r:], zi])
    ab_p = jnp.concatenate(
        [abundances, jnp.zeros((S_PAD - abundances.shape[0],), f32)])
    zeros = jnp.zeros((SROW, L), f32)
    iota = jnp.arange(SROW, dtype=jnp.int32).reshape(RCH, 128)
    t = temperature.astype(f32)
    consts = jnp.concatenate([
        jnp.broadcast_to(jnp.log(t / 300.0), (L,)),
        jnp.broadcast_to(1.0 / t, (L,)),
        jnp.broadcast_to(cr_rate.astype(f32), (L,)),
        jnp.broadcast_to(fuv_rate.astype(f32), (L,)),
    ])

    mesh = plsc.VectorSubcoreMesh(core_axis_name="c", subcore_axis_name="s")
    sc = pl.kernel(
        _sc_body,
        out_type=jax.ShapeDtypeStruct((NC, SROW, L), f32),
        mesh=mesh,
        compiler_params=pltpu.CompilerParams(
            needs_layout_passes=False, use_tc_tiling_on_sc=False),
        scratch_types=[
            pltpu.VMEM((S_PAD,), f32),
            pltpu.VMEM((SROW, L), f32),
            pltpu.VMEM((CH,), f32),
            pltpu.VMEM((CH,), f32),
            pltpu.VMEM((CH,), f32),
            pltpu.VMEM((CH,), f32),
            pltpu.VMEM((CH,), f32),
            pltpu.VMEM((CH,), jnp.int32),
            pltpu.VMEM((CH,), jnp.int32),
            pltpu.VMEM((CH,), jnp.int32),
            pltpu.VMEM((CH,), jnp.int32),
            pltpu.VMEM((RCH, 128), jnp.int32),
            pltpu.VMEM((4 * L,), f32),
            pltpu.VMEM_SHARED((SROW, L), f32),
        ],
    )
    partials = sc(alpha_p, beta_p, gamma_p, zeta_p, xi_p,
                  r1, r2, p1, p2, ab_p, zeros, iota, consts)
    out_pad = pl.pallas_call(
        _combine_body,
        out_shape=jax.ShapeDtypeStruct((160, 128), f32),
    )(partials.reshape(NC, 160, 128))
    return out_pad.reshape(S_PAD)[:NSPEC]


# speedup vs baseline: 339.6393x; 1.6954x over previous
import jax
import jax.numpy as jnp
from jax import lax
from jax.experimental import pallas as pl
from jax.experimental.pallas import tpu as pltpu
from jax.experimental.pallas import tpu_sc as plsc

NC = 2
NS = 16
NW = NC * NS
L = 16

NSPEC = 20000
S_PAD = 20480
SROW = S_PAD // L
RCH = SROW // 128
ORB = SROW // NS
NREAC = 200000
CH = 6256
NV = 392
CHPAD = NV * L
LAST_BASE = NREAC - CH
HEAD = (31 * CH - LAST_BASE) // L


def _sc_body(alpha_h, beta_h, gamma_h, zeta_h, xi_h,
             spec_h, rows_h, ab_h, zeros_h, iota_h, consts_h,
             out_h,
             ab_v, acc_v, al_v, be_v, ga_v, ze_v, xj_v,
             i1_v, i2_v, q1_v, q2_v, iot_v, cv, shared, sem):
    c = lax.axis_index("c")
    s = lax.axis_index("s")
    wid = s * NC + c
    base = lax.min(wid * CH, LAST_BASE)

    cps = [
        pltpu.async_copy(alpha_h.at[pl.ds(base, CH)],
                         al_v.at[pl.ds(0, CH)], sem),
        pltpu.async_copy(beta_h.at[pl.ds(base, CH)],
                         be_v.at[pl.ds(0, CH)], sem),
        pltpu.async_copy(gamma_h.at[pl.ds(base, CH)],
                         ga_v.at[pl.ds(0, CH)], sem),
        pltpu.async_copy(zeta_h.at[pl.ds(base, CH)],
                         ze_v.at[pl.ds(0, CH)], sem),
        pltpu.async_copy(xi_h.at[pl.ds(base, CH)],
                         xj_v.at[pl.ds(0, CH)], sem),
        pltpu.async_copy(spec_h.at[pl.ds(base, CH)],
                         i1_v.at[pl.ds(0, CH)], sem),
        pltpu.async_copy(spec_h.at[pl.ds(NREAC + base, CH)],
                         i2_v.at[pl.ds(0, CH)], sem),
        pltpu.async_copy(rows_h.at[pl.ds(2 * NREAC + base, CH)],
                         q1_v.at[pl.ds(0, CH)], sem),
        pltpu.async_copy(rows_h.at[pl.ds(3 * NREAC + base, CH)],
                         q2_v.at[pl.ds(0, CH)], sem),
        pltpu.async_copy(ab_h, ab_v, sem),
        pltpu.async_copy(iota_h, iot_v, sem),
        pltpu.async_copy(consts_h, cv, sem),
    ]

    @pl.when(s == 0)
    def _zero_shared():
        pltpu.sync_copy(zeros_h, shared)

    zf = jnp.zeros((L,), jnp.float32)

    @plsc.parallel_loop(0, SROW, unroll=8)
    def _zero_acc(j):
        acc_v[j] = zf

    for cp in cps:
        cp.wait()

    zi = jnp.zeros((L,), jnp.int32)
    for ref in (al_v, be_v, ga_v, ze_v, xj_v):
        ref[pl.ds(CH, L)] = zf
    for ref in (i1_v, i2_v, q1_v, q2_v):
        ref[pl.ds(CH, L)] = zi

    @pl.when(wid == NW - 1)
    def _kill_overlap():
        for j in range(HEAD):
            al_v[pl.ds(j * L, L)] = zf
            ze_v[pl.ds(j * L, L)] = zf
            xj_v[pl.ds(j * L, L)] = zf

    c1 = cv[pl.ds(0, L)]
    c2 = cv[pl.ds(L, L)]
    crv = cv[pl.ds(2 * L, L)]
    fuv = cv[pl.ds(3 * L, L)]

    @plsc.parallel_loop(0, NV, unroll=4)
    def _main(j):
        o = j * L
        a = al_v[pl.ds(o, L)]
        b = be_v[pl.ds(o, L)]
        g = ga_v[pl.ds(o, L)]
        z = ze_v[pl.ds(o, L)]
        x = xj_v[pl.ds(o, L)]
        i1 = i1_v[pl.ds(o, L)]
        i2 = i2_v[pl.ds(o, L)]
        q1 = q1_v[pl.ds(o, L)]
        q2 = q2_v[pl.ds(o, L)]
        rate = a * jnp.exp(b * c1 - g * c2) + z * crv + x * fuv
        ab1 = plsc.load_gather(ab_v, [i1])
        ab2 = plsc.load_gather(ab_v, [i2])
        rate = rate * ab1 * ab2
        neg = -rate
        plsc.addupdate_scatter(acc_v, [q1 >> 4, q1 & 15], rate)
        plsc.addupdate_scatter(acc_v, [q2 >> 4, q2 & 15], rate)
        plsc.addupdate_scatter(acc_v, [i1 >> 4, i1 & 15], neg)
        plsc.addupdate_scatter(acc_v, [i2 >> 4, i2 & 15], neg)

    plsc.subcore_barrier()
    for j in range(RCH):
        pltpu.sync_copy(acc_v.at[pl.ds(j * 128, 128)],
                        shared.at[iot_v.at[j]], add=True)
    plsc.subcore_barrier()
    pltpu.sync_copy(shared.at[pl.ds(s * ORB, ORB)],
                    out_h.at[c].at[pl.ds(s * ORB, ORB)])


def _combine_body(x_ref, o_ref):
    o_ref[...] = x_ref[0] + x_ref[1]


def kernel(time, abundances, temperature, cr_rate, fuv_rate, alpha, beta,
           gamma, zeta, xi, inc_vals, pair_reac, pair_species, inc_rows,
           inc_cols):
    f32 = jnp.float32
    zeros = jnp.zeros((SROW, L), f32)
    iota = jnp.arange(SROW, dtype=jnp.int32).reshape(RCH, 128)
    t = temperature.astype(f32)
    consts = jnp.concatenate([
        jnp.broadcast_to(jnp.log(t / 300.0), (L,)),
        jnp.broadcast_to(1.0 / t, (L,)),
        jnp.broadcast_to(cr_rate.astype(f32), (L,)),
        jnp.broadcast_to(fuv_rate.astype(f32), (L,)),
    ])

    mesh = plsc.VectorSubcoreMesh(core_axis_name="c", subcore_axis_name="s")
    sc = pl.kernel(
        _sc_body,
        out_type=jax.ShapeDtypeStruct((NC, SROW, L), f32),
        mesh=mesh,
        compiler_params=pltpu.CompilerParams(
            needs_layout_passes=False, use_tc_tiling_on_sc=False),
        scratch_types=[
            pltpu.VMEM((NSPEC,), f32),
            pltpu.VMEM((SROW, L), f32),
            pltpu.VMEM((CHPAD,), f32),
            pltpu.VMEM((CHPAD,), f32),
            pltpu.VMEM((CHPAD,), f32),
            pltpu.VMEM((CHPAD,), f32),
            pltpu.VMEM((CHPAD,), f32),
            pltpu.VMEM((CHPAD,), jnp.int32),
            pltpu.VMEM((CHPAD,), jnp.int32),
            pltpu.VMEM((CHPAD,), jnp.int32),
            pltpu.VMEM((CHPAD,), jnp.int32),
            pltpu.VMEM((RCH, 128), jnp.int32),
            pltpu.VMEM((4 * L,), f32),
            pltpu.VMEM_SHARED((SROW, L), f32),
            pltpu.SemaphoreType.DMA,
        ],
    )
    partials = sc(alpha, beta, gamma, zeta, xi,
                  pair_species, inc_rows, abundances, zeros, iota, consts)
    out_pad = pl.pallas_call(
        _combine_body,
        out_shape=jax.ShapeDtypeStruct((160, 128), f32),
    )(partials.reshape(NC, 160, 128))
    return out_pad.reshape(S_PAD)[:NSPEC]
